# baseline (speedup 1.0000x reference)
import jax
import jax.numpy as jnp
import numpy as np
from jax import lax
from jax.experimental import pallas as pl
from jax.experimental.pallas import tpu as pltpu
from jax.experimental.pallas import tpu_sc as plsc

_B, _K, _C = 128, 32768, 3
_DROP = 2048
_KEEP = _K - _DROP
_NW = 32
_NC = 2
_BATCHES_PER_W = _B // _NW
_CHUNK = 15360
_NCHUNK = _KEEP // _CHUNK
_PAIRS = _CHUNK // 32
_KEEPW = _KEEP // 2


_ROT_A = (13, 15, 26, 6)
_ROT_B = (17, 29, 16, 24)


def _threefry2x32(k1, k2, x0, x1):
    ks = (np.uint32(k1), np.uint32(k2),
          np.uint32(k1) ^ np.uint32(k2) ^ np.uint32(0x1BD11BDA))
    x0 = x0 + ks[0]
    x1 = x1 + ks[1]
    sched = ((_ROT_A, ks[1], ks[2], 1), (_ROT_B, ks[2], ks[0], 2),
             (_ROT_A, ks[0], ks[1], 3), (_ROT_B, ks[1], ks[2], 4),
             (_ROT_A, ks[2], ks[0], 5))
    for rots, a0, a1, i in sched:
        for r in rots:
            x0 = x0 + x1
            x1 = x0 ^ ((x1 << np.uint32(r)) | (x1 >> np.uint32(32 - r)))
        x0 = x0 + a0
        x1 = x1 + a1 + np.uint32(i)
    return x0, x1


def _split(key, n):
    b1, b2 = _threefry2x32(key[0], key[1], np.zeros(n, np.uint32),
                           np.arange(n, dtype=np.uint32))
    return np.stack([b1, b2], axis=1)


def _permutation(key, n):
    x = np.arange(n, dtype=np.int32)
    for _ in range(2):
        key, sub = _split(key, 2)
        b1, b2 = _threefry2x32(sub[0], sub[1], np.zeros(n, np.uint32),
                               np.arange(n, dtype=np.uint32))
        x = x[np.argsort(b1 ^ b2, kind="stable")]
    return x


def _compute_idx() -> np.ndarray:
    keys = _split(np.array([0, 42], np.uint32), _B)
    return np.stack([_permutation(keys[b], _K)[:_KEEP] for b in range(_B)])


_IDX = _compute_idx()


def _pack_idx16(idx: np.ndarray) -> np.ndarray:
    blocks = idx.reshape(-1, 2, 16)
    packed = np.empty((blocks.shape[0], 32), np.int16)
    packed[:, 0::2] = blocks[:, 0, :]
    packed[:, 1::2] = blocks[:, 1, :]
    return packed.reshape(-1)


_IDX16 = _pack_idx16(_IDX)
_IDX32 = _IDX16.view(np.int32)


def _body(x_hbm, idx_hbm, out_hbm, pl0, pl1, ix0, ix1, ov0, ov1,
          sem_p0, sem_p1, sem_x0, sem_x1, sem_o0, sem_o1):
    wid = lax.axis_index("s") * _NC + lax.axis_index("c")
    plane_v = (pl0, pl1)
    idx_v = (ix0, ix1)
    out_v = (ov0, ov1)
    sem_p = (sem_p0, sem_p1)
    sem_x = (sem_x0, sem_x1)
    sem_o = (sem_o0, sem_o1)

    b0 = wid * _BATCHES_PER_W
    ntasks = 3 * _BATCHES_PER_W

    def plane_load(t):
        return pltpu.async_copy(
            x_hbm.at[t % 3, b0 + t // 3], plane_v[t % 2], sem_p[t % 2])

    def idx_load(j):
        return pltpu.async_copy(
            idx_hbm.at[pl.ds((b0 + j) * _KEEPW, _KEEPW)], idx_v[j % 2],
            sem_x[j % 2])

    pending_idx = [idx_load(0), None]
    pending_plane = [plane_load(0), plane_load(1)]
    pending_out = [None, None]
    out_parity = 0

    for t in range(ntasks):
        j, c = t // 3, t % 3
        b = b0 + j
        if c == 0:
            pending_idx[j % 2].wait()
            if j + 1 < _BATCHES_PER_W:
                pending_idx[(j + 1) % 2] = idx_load(j + 1)
        pending_plane[t % 2].wait()

        for ch in range(_NCHUNK):
            q = out_parity
            out_parity ^= 1
            if pending_out[q] is not None:
                pending_out[q].wait()

            @plsc.parallel_loop(0, _PAIRS, 1, unroll=16)
            def _(k, _q=q, _jp=j % 2, _tp=t % 2, _ch=ch):
                w16 = idx_v[_jp][pl.ds(_ch * (_CHUNK // 2) + k * 16, 16)]
                rows = plsc.unpack(plsc.bitcast(w16, jnp.int16),
                                   format=plsc.PackFormat.INTERLEAVED,
                                   preferred_element_type=jnp.int32)
                for half in range(2):
                    vals = plsc.load_gather(plane_v[_tp], [rows[half]])
                    out_v[_q][pl.ds(k * 32 + half * 16, 16)] = vals

            pending_out[q] = pltpu.async_copy(
                out_v[q],
                out_hbm.at[c, b, pl.ds(ch * _CHUNK, _CHUNK)], sem_o[q])

        if t + 2 < ntasks:
            pending_plane[t % 2] = plane_load(t + 2)

    for q in range(2):
        if pending_out[q] is not None:
            pending_out[q].wait()


@jax.jit
def _gather(xp, idx):
    mesh = plsc.VectorSubcoreMesh(core_axis_name="c", subcore_axis_name="s")
    f = pl.kernel(
        _body,
        out_type=jax.ShapeDtypeStruct((_C, _B, _KEEP), jnp.float32),
        mesh=mesh,
        compiler_params=pltpu.CompilerParams(needs_layout_passes=False),
        scratch_types=[
            pltpu.VMEM((_K,), jnp.float32),
            pltpu.VMEM((_K,), jnp.float32),
            pltpu.VMEM((_KEEPW,), jnp.int32),
            pltpu.VMEM((_KEEPW,), jnp.int32),
            pltpu.VMEM((_CHUNK,), jnp.float32),
            pltpu.VMEM((_CHUNK,), jnp.float32),
            pltpu.SemaphoreType.DMA,
            pltpu.SemaphoreType.DMA,
            pltpu.SemaphoreType.DMA,
            pltpu.SemaphoreType.DMA,
            pltpu.SemaphoreType.DMA,
            pltpu.SemaphoreType.DMA,
        ],
    )
    return f(xp, idx)


def kernel(x):
    xp = jnp.transpose(x, (2, 0, 1))
    op = _gather(xp, jnp.asarray(_IDX32))
    return lax.stop_gradient(jnp.transpose(op, (1, 2, 0)))


# speedup vs baseline: 1.0362x; 1.0362x over previous
import jax
import jax.numpy as jnp
import numpy as np
from jax import lax
from jax.experimental import pallas as pl
from jax.experimental.pallas import tpu as pltpu
from jax.experimental.pallas import tpu_sc as plsc

_B, _K, _C = 128, 32768, 3
_DROP = 2048
_KEEP = _K - _DROP
_NW = 32
_NC = 2
_BATCHES_PER_W = _B // _NW
_CHUNK = 15360
_NCHUNK = _KEEP // _CHUNK
_PAIRS = _CHUNK // 32
_KEEPW = _KEEP // 2


_ROT_A = (13, 15, 26, 6)
_ROT_B = (17, 29, 16, 24)


def _threefry2x32(k1, k2, x0, x1):
    ks = (np.uint32(k1), np.uint32(k2),
          np.uint32(k1) ^ np.uint32(k2) ^ np.uint32(0x1BD11BDA))
    x0 = x0 + ks[0]
    x1 = x1 + ks[1]
    sched = ((_ROT_A, ks[1], ks[2], 1), (_ROT_B, ks[2], ks[0], 2),
             (_ROT_A, ks[0], ks[1], 3), (_ROT_B, ks[1], ks[2], 4),
             (_ROT_A, ks[2], ks[0], 5))
    for rots, a0, a1, i in sched:
        for r in rots:
            x0 = x0 + x1
            x1 = x0 ^ ((x1 << np.uint32(r)) | (x1 >> np.uint32(32 - r)))
        x0 = x0 + a0
        x1 = x1 + a1 + np.uint32(i)
    return x0, x1


def _split(key, n):
    b1, b2 = _threefry2x32(key[0], key[1], np.zeros(n, np.uint32),
                           np.arange(n, dtype=np.uint32))
    return np.stack([b1, b2], axis=1)


def _permutation(key, n):
    x = np.arange(n, dtype=np.int32)
    for _ in range(2):
        key, sub = _split(key, 2)
        b1, b2 = _threefry2x32(sub[0], sub[1], np.zeros(n, np.uint32),
                               np.arange(n, dtype=np.uint32))
        x = x[np.argsort(b1 ^ b2, kind="stable")]
    return x


def _compute_idx() -> np.ndarray:
    keys = _split(np.array([0, 42], np.uint32), _B)
    return np.stack([_permutation(keys[b], _K)[:_KEEP] for b in range(_B)])


_IDX = _compute_idx()


def _pack_idx16(idx: np.ndarray) -> np.ndarray:
    blocks = idx.reshape(-1, 2, 16)
    packed = np.empty((blocks.shape[0], 32), np.int16)
    packed[:, 0::2] = blocks[:, 0, :]
    packed[:, 1::2] = blocks[:, 1, :]
    return packed.reshape(-1)


_IDX16 = _pack_idx16(_IDX)
_IDX32 = _IDX16.view(np.int32)


def _body(x_hbm, idx_hbm, out_hbm, pl0, pl1, ix0, ix1, ov0, ov1,
          sem_p0, sem_p1, sem_x0, sem_x1, sem_o0, sem_o1):
    wid = lax.axis_index("s") * _NC + lax.axis_index("c")
    plane_v = (pl0, pl1)
    idx_v = (ix0, ix1)
    out_v = (ov0, ov1)
    sem_p = (sem_p0, sem_p1)
    sem_x = (sem_x0, sem_x1)
    sem_o = (sem_o0, sem_o1)

    b0 = wid * _BATCHES_PER_W
    ntasks = 3 * _BATCHES_PER_W

    def plane_load(t):
        return pltpu.async_copy(
            x_hbm.at[t % 3, b0 + t // 3], plane_v[t % 2], sem_p[t % 2])

    def idx_load(j):
        return pltpu.async_copy(
            idx_hbm.at[pl.ds((b0 + j) * _KEEPW, _KEEPW)], idx_v[j % 2],
            sem_x[j % 2])

    pending_idx = [idx_load(0), None]
    pending_plane = [plane_load(0), plane_load(1)]
    pending_out = [None, None]
    out_parity = 0

    for t in range(ntasks):
        j, c = t // 3, t % 3
        b = b0 + j
        if c == 0:
            pending_idx[j % 2].wait()
            if j + 1 < _BATCHES_PER_W:
                pending_idx[(j + 1) % 2] = idx_load(j + 1)
        pending_plane[t % 2].wait()

        for ch in range(_NCHUNK):
            q = out_parity
            out_parity ^= 1
            if pending_out[q] is not None:
                pending_out[q].wait()

            @plsc.parallel_loop(0, _PAIRS, 1, unroll=8)
            def _(k, _q=q, _jp=j % 2, _tp=t % 2, _ch=ch):
                w16 = idx_v[_jp][pl.ds(_ch * (_CHUNK // 2) + k * 16, 16)]
                rows = plsc.unpack(plsc.bitcast(w16, jnp.int16),
                                   format=plsc.PackFormat.INTERLEAVED,
                                   preferred_element_type=jnp.int32)
                for half in range(2):
                    vals = plsc.load_gather(plane_v[_tp], [rows[half]])
                    out_v[_q][pl.ds(k * 32 + half * 16, 16)] = vals

            pending_out[q] = pltpu.async_copy(
                out_v[q],
                out_hbm.at[c, b, pl.ds(ch * _CHUNK, _CHUNK)], sem_o[q])

        if t + 2 < ntasks:
            pending_plane[t % 2] = plane_load(t + 2)

    for q in range(2):
        if pending_out[q] is not None:
            pending_out[q].wait()


@jax.jit
def _gather(xp, idx):
    mesh = plsc.VectorSubcoreMesh(core_axis_name="c", subcore_axis_name="s")
    f = pl.kernel(
        _body,
        out_type=jax.ShapeDtypeStruct((_C, _B, _KEEP), jnp.float32),
        mesh=mesh,
        compiler_params=pltpu.CompilerParams(needs_layout_passes=False),
        scratch_types=[
            pltpu.VMEM((_K,), jnp.float32),
            pltpu.VMEM((_K,), jnp.float32),
            pltpu.VMEM((_KEEPW,), jnp.int32),
            pltpu.VMEM((_KEEPW,), jnp.int32),
            pltpu.VMEM((_CHUNK,), jnp.float32),
            pltpu.VMEM((_CHUNK,), jnp.float32),
            pltpu.SemaphoreType.DMA,
            pltpu.SemaphoreType.DMA,
            pltpu.SemaphoreType.DMA,
            pltpu.SemaphoreType.DMA,
            pltpu.SemaphoreType.DMA,
            pltpu.SemaphoreType.DMA,
        ],
    )
    return f(xp, idx)


def kernel(x):
    xp = jnp.transpose(x, (2, 0, 1))
    op = _gather(xp, jnp.asarray(_IDX32))
    return lax.stop_gradient(jnp.transpose(op, (1, 2, 0)))
